# baseline (speedup 1.0000x reference)
import functools
import math

import jax
import jax.numpy as jnp
from jax import lax
from jax.experimental import pallas as pl
from jax.experimental.pallas import tpu as pltpu
from jax.experimental.pallas import tpu_sc as plsc

EPS = 1.1920929e-07
CF = 1.25
COEF = 0.01
NEG = -1e30


def _qkv_body(x_ref, w_ref, g_ref, o_ref):
    x = x_ref[...]
    ms = jnp.mean(x * x, axis=-1, keepdims=True)
    xn = x * lax.rsqrt(ms + EPS) * g_ref[...]
    o_ref[...] = lax.dot_general(xn, w_ref[...], (((1,), (1,)), ((), ())),
                                 preferred_element_type=jnp.float32)


def qkv_call(xs, w, g, tq=256):
    n, c = xs.shape
    c3 = w.shape[0]
    return pl.pallas_call(
        _qkv_body,
        grid=(n // tq,),
        in_specs=[
            pl.BlockSpec((tq, c), lambda i: (i, 0)),
            pl.BlockSpec((c3, c), lambda i: (0, 0)),
            pl.BlockSpec((1, c), lambda i: (0, 0)),
        ],
        out_specs=pl.BlockSpec((tq, c3), lambda i: (i, 0)),
        out_shape=jax.ShapeDtypeStruct((n, c3), jnp.float32),
    )(xs, w, g)


def _attn_body(q_ref, k_ref, v_ref, o_ref, s_ref, *, tq, hd, scale):
    i = pl.program_id(1)
    nsub = q_ref.shape[1] // hd
    outs = []
    for sub in range(nsub):
        q = q_ref[:, sub * hd:(sub + 1) * hd]
        s_ref[...] = jnp.full(s_ref.shape, NEG, jnp.float32)

        def body(j, carry):
            k = k_ref[pl.ds(j * tq, tq), sub * hd:(sub + 1) * hd]
            s = lax.dot_general(q, k, (((1,), (1,)), ((), ())),
                                preferred_element_type=jnp.float32) * scale
            rows = lax.broadcasted_iota(jnp.int32, s.shape, 0) + i * tq
            cols = lax.broadcasted_iota(jnp.int32, s.shape, 1) + j * tq
            s_ref[:, pl.ds(j * tq, tq)] = jnp.where(cols <= rows, s, NEG)
            return carry

        lax.fori_loop(0, i + 1, body, 0)
        s = s_ref[...]
        m = jnp.max(s, axis=-1, keepdims=True)
        p = jnp.exp(s - m)
        p = p / jnp.sum(p, axis=-1, keepdims=True)
        outs.append(jnp.dot(p, v_ref[:, sub * hd:(sub + 1) * hd],
                            preferred_element_type=jnp.float32))
    o_ref[...] = jnp.concatenate(outs, axis=-1)


def attn_call(qkv, nh, hd, tq=256, hpb=2):
    n = qkv.shape[0]
    c = nh * hd
    w = hpb * hd
    cb = c // w
    scale = 1.0 / math.sqrt(hd)
    return pl.pallas_call(
        functools.partial(_attn_body, tq=tq, hd=hd, scale=scale),
        grid=(nh // hpb, n // tq),
        in_specs=[
            pl.BlockSpec((tq, w), lambda hh, i: (i, hh)),
            pl.BlockSpec((n, w), lambda hh, i: (0, cb + hh)),
            pl.BlockSpec((n, w), lambda hh, i: (0, 2 * cb + hh)),
        ],
        out_specs=pl.BlockSpec((tq, w), lambda hh, i: (i, hh)),
        out_shape=jax.ShapeDtypeStruct((n, c), jnp.float32),
        scratch_shapes=[pltpu.VMEM((tq, n), jnp.float32)],
    )(qkv, qkv, qkv)


def _proj_body(y_ref, x_ref, w_ref, g_ref, gw_ref, h_ref, xf_ref, lg_ref):
    h = x_ref[...] + lax.dot_general(y_ref[...], w_ref[...],
                                     (((1,), (1,)), ((), ())),
                                     preferred_element_type=jnp.float32)
    h_ref[...] = h
    ms = jnp.mean(h * h, axis=-1, keepdims=True)
    xf = h * lax.rsqrt(ms + EPS) * g_ref[...]
    xf_ref[...] = xf
    lg_ref[...] = lax.dot_general(xf, gw_ref[...], (((1,), (1,)), ((), ())),
                                  preferred_element_type=jnp.float32)


def proj_call(y, xs, w, g, gw_pad, tq=256):
    n, c = xs.shape
    el = gw_pad.shape[0]
    return pl.pallas_call(
        _proj_body,
        grid=(n // tq,),
        in_specs=[
            pl.BlockSpec((tq, c), lambda i: (i, 0)),
            pl.BlockSpec((tq, c), lambda i: (i, 0)),
            pl.BlockSpec((c, c), lambda i: (0, 0)),
            pl.BlockSpec((1, c), lambda i: (0, 0)),
            pl.BlockSpec((el, c), lambda i: (0, 0)),
        ],
        out_specs=[
            pl.BlockSpec((tq, c), lambda i: (i, 0)),
            pl.BlockSpec((tq, c), lambda i: (i, 0)),
            pl.BlockSpec((tq, el), lambda i: (i, 0)),
        ],
        out_shape=[
            jax.ShapeDtypeStruct((n, c), jnp.float32),
            jax.ShapeDtypeStruct((n, c), jnp.float32),
            jax.ShapeDtypeStruct((n, el), jnp.float32),
        ],
    )(y, xs, w, g, gw_pad)


def _route_body(lg_ref, s0_ref, s1_ref, w0_ref, w1_ref, loss_ref, *,
                e, capacity, dump):
    lg = lg_ref[...]
    n = lg.shape[0]
    lanes = lax.broadcasted_iota(jnp.int32, lg.shape, 1)
    lgm = jnp.where(lanes < e, lg, NEG)
    m0 = jnp.max(lgm, axis=-1, keepdims=True)
    e0 = jnp.min(jnp.where(lgm == m0, lanes, 127), axis=-1, keepdims=True)
    lg2 = jnp.where(lanes == e0, NEG, lgm)
    m1 = jnp.max(lg2, axis=-1, keepdims=True)
    e1 = jnp.min(jnp.where(lg2 == m1, lanes, 127), axis=-1, keepdims=True)

    ex = jnp.where(lanes < e, jnp.exp(lgm - m0), 0.0)
    probs = ex / jnp.sum(ex, axis=-1, keepdims=True)
    p0 = jnp.sum(jnp.where(lanes == e0, probs, 0.0), axis=-1, keepdims=True)
    p1 = jnp.sum(jnp.where(lanes == e1, probs, 0.0), axis=-1, keepdims=True)
    denom = p0 + p1 + 1e-9
    w0 = p0 / denom
    w1 = p1 / denom

    is0 = (lanes == e0).astype(jnp.float32)
    is1 = (lanes == e1).astype(jnp.float32)
    cnt = is0 + is1
    c = cnt
    sft = 1
    while sft < n:
        c = c + jnp.concatenate([jnp.zeros((sft, c.shape[1]), c.dtype),
                                 c[:-sft]], axis=0)
        sft *= 2
    cume = c - cnt
    pos0 = jnp.sum(jnp.where(lanes == e0, cume, 0.0), axis=-1, keepdims=True)
    pos1 = jnp.sum(jnp.where(lanes == e1, cume, 0.0), axis=-1, keepdims=True)
    pos0 = pos0.astype(jnp.int32)
    pos1 = pos1.astype(jnp.int32)
    v0 = pos0 < capacity
    v1 = pos1 < capacity
    s0_ref[...] = jnp.where(v0, e0 * capacity + pos0, dump)
    s1_ref[...] = jnp.where(v1, e1 * capacity + pos1, dump)
    w0_ref[...] = jnp.where(v0, w0, 0.0)
    w1_ref[...] = jnp.where(v1, w1, 0.0)

    cnt_top1 = jnp.sum(is0, axis=0, keepdims=True)
    load_frac = cnt_top1 / (jnp.sum(cnt_top1) + 1e-9)
    importance = jnp.sum(probs, axis=0, keepdims=True) / (jnp.sum(probs) + 1e-9)
    loss_ref[...] = (COEF * e * jnp.sum(load_frac * importance)).reshape(1, 1)


def route_call(lg, e, capacity, dump):
    n, el = lg.shape
    return pl.pallas_call(
        functools.partial(_route_body, e=e, capacity=capacity, dump=dump),
        in_specs=[pl.BlockSpec((n, el), lambda: (0, 0))],
        out_specs=[
            pl.BlockSpec((n, 1), lambda: (0, 0)),
            pl.BlockSpec((n, 1), lambda: (0, 0)),
            pl.BlockSpec((n, 1), lambda: (0, 0)),
            pl.BlockSpec((n, 1), lambda: (0, 0)),
            pl.BlockSpec((1, 1), lambda: (0, 0)),
        ],
        out_shape=[
            jax.ShapeDtypeStruct((n, 1), jnp.int32),
            jax.ShapeDtypeStruct((n, 1), jnp.int32),
            jax.ShapeDtypeStruct((n, 1), jnp.float32),
            jax.ShapeDtypeStruct((n, 1), jnp.float32),
            jax.ShapeDtypeStruct((1, 1), jnp.float32),
        ],
    )(lg)


def _ffn_body(x_ref, w1_ref, b1_ref, w2_ref, b2_ref, o_ref):
    a = lax.dot_general(x_ref[...], w1_ref[0], (((1,), (1,)), ((), ())),
                        preferred_element_type=jnp.float32) + b1_ref[0]
    a = a * jax.nn.sigmoid(a)
    o_ref[...] = lax.dot_general(a, w2_ref[0], (((1,), (1,)), ((), ())),
                                 preferred_element_type=jnp.float32) \
        + b2_ref[0]


def ffn_call(disp, w1, b1, w2, b2, e, cap, nd):
    _, c = disp.shape
    hh = w1.shape[1]
    return pl.pallas_call(
        _ffn_body,
        grid=(e,),
        in_specs=[
            pl.BlockSpec((cap, c), lambda ei: (ei, 0)),
            pl.BlockSpec((1, hh, c), lambda ei: (ei, 0, 0)),
            pl.BlockSpec((1, 1, hh), lambda ei: (ei, 0, 0)),
            pl.BlockSpec((1, c, hh), lambda ei: (ei, 0, 0)),
            pl.BlockSpec((1, 1, c), lambda ei: (ei, 0, 0)),
        ],
        out_specs=pl.BlockSpec((cap, c), lambda ei: (ei, 0)),
        out_shape=jax.ShapeDtypeStruct((nd, c), jnp.float32),
    )(disp, w1, b1, w2, b2)


def _comb_body(h_ref, b0_ref, b1_ref, w0_ref, w1_ref, o_ref):
    w0 = w0_ref[...]
    w1 = w1_ref[...]
    t0 = jnp.where(w0 != 0.0, w0 * b0_ref[...], 0.0)
    t1 = jnp.where(w1 != 0.0, w1 * b1_ref[...], 0.0)
    o_ref[...] = h_ref[...] + t0 + t1


def comb_call(h, b0, b1, w0, w1, tq=256):
    n, c = h.shape
    return pl.pallas_call(
        _comb_body,
        grid=(n // tq,),
        in_specs=[
            pl.BlockSpec((tq, c), lambda i: (i, 0)),
            pl.BlockSpec((tq, c), lambda i: (i, 0)),
            pl.BlockSpec((tq, c), lambda i: (i, 0)),
            pl.BlockSpec((tq, 1), lambda i: (i, 0)),
            pl.BlockSpec((tq, 1), lambda i: (i, 0)),
        ],
        out_specs=pl.BlockSpec((tq, c), lambda i: (i, 0)),
        out_shape=jax.ShapeDtypeStruct((n, c), jnp.float32),
    )(h, b0, b1, w0, w1)


def dispatch_call(xf, s0, s1, nd):
    n, c = xf.shape
    info = plsc.get_sparse_core_info()
    nc, ns = info.num_cores, info.num_subcores
    nw = nc * ns
    nt = n // nw
    ch = min(nt, 64)
    mesh = plsc.VectorSubcoreMesh(core_axis_name="c", subcore_axis_name="s")

    @functools.partial(
        pl.kernel, mesh=mesh,
        out_type=jax.ShapeDtypeStruct((nd, c), jnp.float32),
        scratch_types=[
            pltpu.VMEM((ch,), jnp.int32),
            pltpu.VMEM((ch,), jnp.int32),
            pltpu.VMEM((ch, c), jnp.float32),
            pltpu.SemaphoreType.DMA,
            pltpu.SemaphoreType.DMA,
            pltpu.SemaphoreType.DMA,
        ],
    )
    def k(xf_hbm, s0_hbm, s1_hbm, out_hbm, i0_v, i1_v, rows_v, sem0, sem1,
          sem2):
        wid = lax.axis_index("s") * nc + lax.axis_index("c")
        for ci in range(nt // ch):
            base = wid * nt + ci * ch
            c0 = pltpu.async_copy(s0_hbm.at[pl.ds(base, ch)], i0_v, sem0)
            c1 = pltpu.async_copy(s1_hbm.at[pl.ds(base, ch)], i1_v, sem1)
            c2 = pltpu.async_copy(xf_hbm.at[pl.ds(base, ch)], rows_v, sem2)
            c0.wait()
            c2.wait()
            d0 = pltpu.async_copy(rows_v, out_hbm.at[i0_v], sem0)
            c1.wait()
            d1 = pltpu.async_copy(rows_v, out_hbm.at[i1_v], sem1)
            d0.wait()
            d1.wait()

    return k(xf, s0, s1)


def combine_gather_call(src, s0, s1, n):
    nd, c = src.shape
    info = plsc.get_sparse_core_info()
    nc, ns = info.num_cores, info.num_subcores
    nw = nc * ns
    nt = n // nw
    ch = min(nt, 32)
    nu = 2 * (nt // ch)
    mesh = plsc.VectorSubcoreMesh(core_axis_name="c", subcore_axis_name="s")

    @functools.partial(
        pl.kernel, mesh=mesh,
        out_type=[jax.ShapeDtypeStruct((n, c), jnp.float32),
                  jax.ShapeDtypeStruct((n, c), jnp.float32)],
        scratch_types=[
            pltpu.VMEM((nt,), jnp.int32),
            pltpu.VMEM((nt,), jnp.int32),
            pltpu.VMEM((ch, c), jnp.float32),
            pltpu.VMEM((ch, c), jnp.float32),
            pltpu.SemaphoreType.DMA,
            pltpu.SemaphoreType.DMA,
            pltpu.SemaphoreType.DMA,
            pltpu.SemaphoreType.DMA,
        ],
    )
    def k(src_hbm, s0_hbm, s1_hbm, b0_hbm, b1_hbm, i0_v, i1_v, ra_v, rb_v,
          sa, sb, swa, swb):
        wid = lax.axis_index("s") * nc + lax.axis_index("c")
        base = wid * nt
        c0 = pltpu.async_copy(s0_hbm.at[pl.ds(base, nt)], i0_v, sa)
        c1 = pltpu.async_copy(s1_hbm.at[pl.ds(base, nt)], i1_v, sb)
        c0.wait()
        c1.wait()
        units = []
        for ci in range(nt // ch):
            units.append((i0_v.at[pl.ds(ci * ch, ch)], b0_hbm, base + ci * ch))
        for ci in range(nt // ch):
            units.append((i1_v.at[pl.ds(ci * ch, ch)], b1_hbm, base + ci * ch))
        bufs = [(ra_v, sa), (rb_v, sb)]
        wsems = [swa, swb]
        gathers = [None] * nu
        writes = [None] * nu

        def fire_write(u):
            buf, _ = bufs[u % 2]
            _, dst, off = units[u]
            gathers[u].wait()
            writes[u] = pltpu.async_copy(buf, dst.at[pl.ds(off, ch)],
                                         wsems[u % 2])

        for u in range(nu):
            buf, gsem = bufs[u % 2]
            if u >= 2:
                writes[u - 2].wait()
            idx, _, _ = units[u]
            gathers[u] = pltpu.async_copy(src_hbm.at[idx], buf, gsem)
            if u >= 1:
                fire_write(u - 1)
        fire_write(nu - 1)
        writes[nu - 2].wait()
        writes[nu - 1].wait()

    return k(src, s0, s1)


def kernel(x, ln1_w, ln2_w, c_attn_w, c_proj_w, gate_w, fc1_w, fc1_b,
           fc2_w, fc2_b):
    b, t, c = x.shape
    n = b * t
    nh = 16
    hd = c // nh
    e = gate_w.shape[0]
    h_dim = fc1_w.shape[1]
    capacity = max(int(math.ceil(CF * n / e)), 1)
    nd = e * capacity + 8
    dump = e * capacity
    hp = ((h_dim + 255) // 256) * 256
    el = 128

    xs = x.reshape(n, c)
    gw_pad = jnp.pad(gate_w, ((0, el - e), (0, 0)))

    qkv = qkv_call(xs, c_attn_w, ln1_w.reshape(1, c))
    y = attn_call(qkv, nh, hd)
    h, xf, lg = proj_call(y, xs, c_proj_w, ln2_w.reshape(1, c), gw_pad)
    s0, s1, w0, w1, loss = route_call(lg, e, capacity, dump)
    s0f = s0.reshape(n)
    s1f = s1.reshape(n)
    disp = dispatch_call(xf, s0f, s1f, nd)
    f2 = ffn_call(disp, fc1_w, fc1_b.reshape(e, 1, h_dim), fc2_w,
                  fc2_b.reshape(e, 1, c), e, capacity, nd)
    b0, b1 = combine_gather_call(f2, s0f, s1f, n)
    out = comb_call(h, b0, b1, w0, w1)
    return out.reshape(b, t, c), loss[0, 0]


# speedup vs baseline: 1.1373x; 1.1373x over previous
import functools
import math

import jax
import jax.numpy as jnp
from jax import lax
from jax.experimental import pallas as pl
from jax.experimental.pallas import tpu as pltpu
from jax.experimental.pallas import tpu_sc as plsc

EPS = 1.1920929e-07
CF = 1.25
COEF = 0.01
NEG = -1e30


def _qkv_body(x_ref, w_ref, g_ref, o_ref):
    x = x_ref[...]
    ms = jnp.mean(x * x, axis=-1, keepdims=True)
    xn = x * lax.rsqrt(ms + EPS) * g_ref[...]
    o_ref[...] = lax.dot_general(xn, w_ref[...], (((1,), (1,)), ((), ())),
                                 preferred_element_type=jnp.float32)


def qkv_call(xs, w, g, tq=256):
    n, c = xs.shape
    c3 = w.shape[0]
    return pl.pallas_call(
        _qkv_body,
        grid=(n // tq,),
        in_specs=[
            pl.BlockSpec((tq, c), lambda i: (i, 0)),
            pl.BlockSpec((c3, c), lambda i: (0, 0)),
            pl.BlockSpec((1, c), lambda i: (0, 0)),
        ],
        out_specs=pl.BlockSpec((tq, c3), lambda i: (i, 0)),
        out_shape=jax.ShapeDtypeStruct((n, c3), jnp.float32),
    )(xs, w, g)


def _attn_body(q_ref, k_ref, v_ref, o_ref, *, tq, hd, scale):
    i = pl.program_id(1)
    outs = []
    for sub in range(q_ref.shape[1] // hd):
        q = q_ref[:, sub * hd:(sub + 1) * hd]
        k = k_ref[:, sub * hd:(sub + 1) * hd]
        s = lax.dot_general(q, k, (((1,), (1,)), ((), ())),
                            preferred_element_type=jnp.float32) * scale
        rows = lax.broadcasted_iota(jnp.int32, s.shape, 0) + i * tq
        cols = lax.broadcasted_iota(jnp.int32, s.shape, 1)
        s = jnp.where(cols <= rows, s, NEG)
        m = jnp.max(s, axis=-1, keepdims=True)
        p = jnp.exp(s - m)
        p = p / jnp.sum(p, axis=-1, keepdims=True)
        outs.append(jnp.dot(p, v_ref[:, sub * hd:(sub + 1) * hd],
                            preferred_element_type=jnp.float32))
    o_ref[...] = jnp.concatenate(outs, axis=-1)


def attn_call(qkv, nh, hd, tq=256, hpb=2):
    n = qkv.shape[0]
    c = nh * hd
    w = hpb * hd
    cb = c // w
    scale = 1.0 / math.sqrt(hd)
    return pl.pallas_call(
        functools.partial(_attn_body, tq=tq, hd=hd, scale=scale),
        grid=(nh // hpb, n // tq),
        in_specs=[
            pl.BlockSpec((tq, w), lambda hh, i: (i, hh)),
            pl.BlockSpec((n, w), lambda hh, i: (0, cb + hh)),
            pl.BlockSpec((n, w), lambda hh, i: (0, 2 * cb + hh)),
        ],
        out_specs=pl.BlockSpec((tq, w), lambda hh, i: (i, hh)),
        out_shape=jax.ShapeDtypeStruct((n, c), jnp.float32),
    )(qkv, qkv, qkv)


def _proj_body(y_ref, x_ref, w_ref, g_ref, gw_ref, h_ref, xf_ref, lg_ref):
    h = x_ref[...] + lax.dot_general(y_ref[...], w_ref[...],
                                     (((1,), (1,)), ((), ())),
                                     preferred_element_type=jnp.float32)
    h_ref[...] = h
    ms = jnp.mean(h * h, axis=-1, keepdims=True)
    xf = h * lax.rsqrt(ms + EPS) * g_ref[...]
    xf_ref[...] = xf
    lg_ref[...] = lax.dot_general(xf, gw_ref[...], (((1,), (1,)), ((), ())),
                                  preferred_element_type=jnp.float32)


def proj_call(y, xs, w, g, gw_pad, tq=256):
    n, c = xs.shape
    el = gw_pad.shape[0]
    return pl.pallas_call(
        _proj_body,
        grid=(n // tq,),
        in_specs=[
            pl.BlockSpec((tq, c), lambda i: (i, 0)),
            pl.BlockSpec((tq, c), lambda i: (i, 0)),
            pl.BlockSpec((c, c), lambda i: (0, 0)),
            pl.BlockSpec((1, c), lambda i: (0, 0)),
            pl.BlockSpec((el, c), lambda i: (0, 0)),
        ],
        out_specs=[
            pl.BlockSpec((tq, c), lambda i: (i, 0)),
            pl.BlockSpec((tq, c), lambda i: (i, 0)),
            pl.BlockSpec((tq, el), lambda i: (i, 0)),
        ],
        out_shape=[
            jax.ShapeDtypeStruct((n, c), jnp.float32),
            jax.ShapeDtypeStruct((n, c), jnp.float32),
            jax.ShapeDtypeStruct((n, el), jnp.float32),
        ],
    )(y, xs, w, g, gw_pad)


def _route_body(lg_ref, s0_ref, s1_ref, w0_ref, w1_ref, loss_ref, *,
                e, capacity, dump):
    lg = lg_ref[...]
    n = lg.shape[0]
    lanes = lax.broadcasted_iota(jnp.int32, lg.shape, 1)
    lgm = jnp.where(lanes < e, lg, NEG)
    m0 = jnp.max(lgm, axis=-1, keepdims=True)
    e0 = jnp.min(jnp.where(lgm == m0, lanes, 127), axis=-1, keepdims=True)
    lg2 = jnp.where(lanes == e0, NEG, lgm)
    m1 = jnp.max(lg2, axis=-1, keepdims=True)
    e1 = jnp.min(jnp.where(lg2 == m1, lanes, 127), axis=-1, keepdims=True)

    ex = jnp.where(lanes < e, jnp.exp(lgm - m0), 0.0)
    probs = ex / jnp.sum(ex, axis=-1, keepdims=True)
    p0 = jnp.sum(jnp.where(lanes == e0, probs, 0.0), axis=-1, keepdims=True)
    p1 = jnp.sum(jnp.where(lanes == e1, probs, 0.0), axis=-1, keepdims=True)
    denom = p0 + p1 + 1e-9
    w0 = p0 / denom
    w1 = p1 / denom

    is0 = (lanes == e0).astype(jnp.float32)
    is1 = (lanes == e1).astype(jnp.float32)
    cnt = is0 + is1
    c = cnt
    sft = 1
    while sft < n:
        c = c + jnp.concatenate([jnp.zeros((sft, c.shape[1]), c.dtype),
                                 c[:-sft]], axis=0)
        sft *= 2
    cume = c - cnt
    pos0 = jnp.sum(jnp.where(lanes == e0, cume, 0.0), axis=-1, keepdims=True)
    pos1 = jnp.sum(jnp.where(lanes == e1, cume, 0.0), axis=-1, keepdims=True)
    pos0 = pos0.astype(jnp.int32)
    pos1 = pos1.astype(jnp.int32)
    v0 = pos0 < capacity
    v1 = pos1 < capacity
    s0_ref[...] = jnp.where(v0, e0 * capacity + pos0, dump)
    s1_ref[...] = jnp.where(v1, e1 * capacity + pos1, dump)
    w0_ref[...] = jnp.where(v0, w0, 0.0)
    w1_ref[...] = jnp.where(v1, w1, 0.0)

    cnt_top1 = jnp.sum(is0, axis=0, keepdims=True)
    load_frac = cnt_top1 / (jnp.sum(cnt_top1) + 1e-9)
    importance = jnp.sum(probs, axis=0, keepdims=True) / (jnp.sum(probs) + 1e-9)
    loss_ref[...] = (COEF * e * jnp.sum(load_frac * importance)).reshape(1, 1)


def route_call(lg, e, capacity, dump):
    n, el = lg.shape
    return pl.pallas_call(
        functools.partial(_route_body, e=e, capacity=capacity, dump=dump),
        in_specs=[pl.BlockSpec((n, el), lambda: (0, 0))],
        out_specs=[
            pl.BlockSpec((n, 1), lambda: (0, 0)),
            pl.BlockSpec((n, 1), lambda: (0, 0)),
            pl.BlockSpec((n, 1), lambda: (0, 0)),
            pl.BlockSpec((n, 1), lambda: (0, 0)),
            pl.BlockSpec((1, 1), lambda: (0, 0)),
        ],
        out_shape=[
            jax.ShapeDtypeStruct((n, 1), jnp.int32),
            jax.ShapeDtypeStruct((n, 1), jnp.int32),
            jax.ShapeDtypeStruct((n, 1), jnp.float32),
            jax.ShapeDtypeStruct((n, 1), jnp.float32),
            jax.ShapeDtypeStruct((1, 1), jnp.float32),
        ],
    )(lg)


def _ffn_body(x_ref, w1_ref, b1_ref, w2_ref, b2_ref, o_ref):
    a = lax.dot_general(x_ref[...], w1_ref[0], (((1,), (1,)), ((), ())),
                        preferred_element_type=jnp.float32) + b1_ref[0]
    a = a * jax.nn.sigmoid(a)
    o_ref[...] = lax.dot_general(a, w2_ref[0], (((1,), (1,)), ((), ())),
                                 preferred_element_type=jnp.float32) \
        + b2_ref[0]


def ffn_call(disp, w1, b1, w2, b2, e, cap, nd):
    _, c = disp.shape
    hh = w1.shape[1]
    return pl.pallas_call(
        _ffn_body,
        grid=(e,),
        in_specs=[
            pl.BlockSpec((cap, c), lambda ei: (ei, 0)),
            pl.BlockSpec((1, hh, c), lambda ei: (ei, 0, 0)),
            pl.BlockSpec((1, 1, hh), lambda ei: (ei, 0, 0)),
            pl.BlockSpec((1, c, hh), lambda ei: (ei, 0, 0)),
            pl.BlockSpec((1, 1, c), lambda ei: (ei, 0, 0)),
        ],
        out_specs=pl.BlockSpec((cap, c), lambda ei: (ei, 0)),
        out_shape=jax.ShapeDtypeStruct((nd, c), jnp.float32),
    )(disp, w1, b1, w2, b2)


def _comb_body(h_ref, b0_ref, b1_ref, w0_ref, w1_ref, o_ref):
    w0 = w0_ref[...]
    w1 = w1_ref[...]
    t0 = jnp.where(w0 != 0.0, w0 * b0_ref[...], 0.0)
    t1 = jnp.where(w1 != 0.0, w1 * b1_ref[...], 0.0)
    o_ref[...] = h_ref[...] + t0 + t1


def comb_call(h, b0, b1, w0, w1, tq=256):
    n, c = h.shape
    return pl.pallas_call(
        _comb_body,
        grid=(n // tq,),
        in_specs=[
            pl.BlockSpec((tq, c), lambda i: (i, 0)),
            pl.BlockSpec((tq, c), lambda i: (i, 0)),
            pl.BlockSpec((tq, c), lambda i: (i, 0)),
            pl.BlockSpec((tq, 1), lambda i: (i, 0)),
            pl.BlockSpec((tq, 1), lambda i: (i, 0)),
        ],
        out_specs=pl.BlockSpec((tq, c), lambda i: (i, 0)),
        out_shape=jax.ShapeDtypeStruct((n, c), jnp.float32),
    )(h, b0, b1, w0, w1)


def dispatch_call(xf, s0, s1, nd):
    n, c = xf.shape
    info = plsc.get_sparse_core_info()
    nc, ns = info.num_cores, info.num_subcores
    nw = nc * ns
    nt = n // nw
    ch = min(nt, 64)
    mesh = plsc.VectorSubcoreMesh(core_axis_name="c", subcore_axis_name="s")

    @functools.partial(
        pl.kernel, mesh=mesh,
        out_type=jax.ShapeDtypeStruct((nd, c), jnp.float32),
        scratch_types=[
            pltpu.VMEM((ch,), jnp.int32),
            pltpu.VMEM((ch,), jnp.int32),
            pltpu.VMEM((ch, c), jnp.float32),
            pltpu.SemaphoreType.DMA,
            pltpu.SemaphoreType.DMA,
            pltpu.SemaphoreType.DMA,
        ],
    )
    def k(xf_hbm, s0_hbm, s1_hbm, out_hbm, i0_v, i1_v, rows_v, sem0, sem1,
          sem2):
        wid = lax.axis_index("s") * nc + lax.axis_index("c")
        for ci in range(nt // ch):
            base = wid * nt + ci * ch
            c0 = pltpu.async_copy(s0_hbm.at[pl.ds(base, ch)], i0_v, sem0)
            c1 = pltpu.async_copy(s1_hbm.at[pl.ds(base, ch)], i1_v, sem1)
            c2 = pltpu.async_copy(xf_hbm.at[pl.ds(base, ch)], rows_v, sem2)
            c0.wait()
            c2.wait()
            d0 = pltpu.async_copy(rows_v, out_hbm.at[i0_v], sem0)
            c1.wait()
            d1 = pltpu.async_copy(rows_v, out_hbm.at[i1_v], sem1)
            d0.wait()
            d1.wait()

    return k(xf, s0, s1)


def combine_gather_call(src, s0, s1, n):
    nd, c = src.shape
    info = plsc.get_sparse_core_info()
    nc, ns = info.num_cores, info.num_subcores
    nw = nc * ns
    nt = n // nw
    ch = min(nt, 32)
    nu = 2 * (nt // ch)
    mesh = plsc.VectorSubcoreMesh(core_axis_name="c", subcore_axis_name="s")

    @functools.partial(
        pl.kernel, mesh=mesh,
        out_type=[jax.ShapeDtypeStruct((n, c), jnp.float32),
                  jax.ShapeDtypeStruct((n, c), jnp.float32)],
        scratch_types=[
            pltpu.VMEM((nt,), jnp.int32),
            pltpu.VMEM((nt,), jnp.int32),
            pltpu.VMEM((ch, c), jnp.float32),
            pltpu.VMEM((ch, c), jnp.float32),
            pltpu.SemaphoreType.DMA,
            pltpu.SemaphoreType.DMA,
            pltpu.SemaphoreType.DMA,
            pltpu.SemaphoreType.DMA,
        ],
    )
    def k(src_hbm, s0_hbm, s1_hbm, b0_hbm, b1_hbm, i0_v, i1_v, ra_v, rb_v,
          sa, sb, swa, swb):
        wid = lax.axis_index("s") * nc + lax.axis_index("c")
        base = wid * nt
        c0 = pltpu.async_copy(s0_hbm.at[pl.ds(base, nt)], i0_v, sa)
        c1 = pltpu.async_copy(s1_hbm.at[pl.ds(base, nt)], i1_v, sb)
        c0.wait()
        c1.wait()
        units = []
        for ci in range(nt // ch):
            units.append((i0_v.at[pl.ds(ci * ch, ch)], b0_hbm, base + ci * ch))
        for ci in range(nt // ch):
            units.append((i1_v.at[pl.ds(ci * ch, ch)], b1_hbm, base + ci * ch))
        bufs = [(ra_v, sa), (rb_v, sb)]
        wsems = [swa, swb]
        gathers = [None] * nu
        writes = [None] * nu

        def fire_write(u):
            buf, _ = bufs[u % 2]
            _, dst, off = units[u]
            gathers[u].wait()
            writes[u] = pltpu.async_copy(buf, dst.at[pl.ds(off, ch)],
                                         wsems[u % 2])

        for u in range(nu):
            buf, gsem = bufs[u % 2]
            if u >= 2:
                writes[u - 2].wait()
            idx, _, _ = units[u]
            gathers[u] = pltpu.async_copy(src_hbm.at[idx], buf, gsem)
            if u >= 1:
                fire_write(u - 1)
        fire_write(nu - 1)
        writes[nu - 2].wait()
        writes[nu - 1].wait()

    return k(src, s0, s1)


def kernel(x, ln1_w, ln2_w, c_attn_w, c_proj_w, gate_w, fc1_w, fc1_b,
           fc2_w, fc2_b):
    b, t, c = x.shape
    n = b * t
    nh = 16
    hd = c // nh
    e = gate_w.shape[0]
    h_dim = fc1_w.shape[1]
    capacity = max(int(math.ceil(CF * n / e)), 1)
    nd = e * capacity + 8
    dump = e * capacity
    hp = ((h_dim + 255) // 256) * 256
    el = 128

    xs = x.reshape(n, c)
    gw_pad = jnp.pad(gate_w, ((0, el - e), (0, 0)))

    qkv = qkv_call(xs, c_attn_w, ln1_w.reshape(1, c))
    y = attn_call(qkv, nh, hd)
    h, xf, lg = proj_call(y, xs, c_proj_w, ln2_w.reshape(1, c), gw_pad)
    s0, s1, w0, w1, loss = route_call(lg, e, capacity, dump)
    s0f = s0.reshape(n)
    s1f = s1.reshape(n)
    disp = dispatch_call(xf, s0f, s1f, nd)
    f2 = ffn_call(disp, fc1_w, fc1_b.reshape(e, 1, h_dim), fc2_w,
                  fc2_b.reshape(e, 1, c), e, capacity, nd)
    b0, b1 = combine_gather_call(f2, s0f, s1f, n)
    out = comb_call(h, b0, b1, w0, w1)
    return out.reshape(b, t, c), loss[0, 0]
